# Initial kernel scaffold; baseline (speedup 1.0000x reference)
import jax
import jax.numpy as jnp
from jax.experimental import pallas as pl


def kernel(feature, mask_intra, umask, W_init_trans, b_init_trans, W_qinter, b_qinter, W_attn, b_attn, W_out, b_out):
    raise NotImplementedError("write your pallas kernel here")


# speedup vs baseline: 2.2727x; 2.2727x over previous
import jax
import jax.numpy as jnp
from jax.experimental import pallas as pl
from jax.experimental.pallas import tpu as pltpu

B, T, D = 4, 128, 512
_F32 = jnp.float32
_CONTRACT_LAST = (((1,), (1,)), ((), ()))
_CONTRACT_NATIVE = (((1,), (0,)), ((), ()))


def _sse_body(x_ref, mi_ref, um_ref, wit_ref, bit_ref, wq_ref, bq_ref,
              wa_ref, wo_ref, bo_ref, out_ref,
              v_ref, ft_ref, qw_ref):
    x = x_ref[:]
    feat = jnp.maximum(
        jax.lax.dot_general(x, wit_ref[:], _CONTRACT_LAST,
                            preferred_element_type=_F32) + bit_ref[:], 0.0)
    ft_ref[:] = feat
    q = jax.lax.dot_general(feat, wq_ref[:], _CONTRACT_LAST,
                            preferred_element_type=_F32) + bq_ref[:]
    qw_ref[:] = q * wa_ref[:]

    lane = jax.lax.broadcasted_iota(jnp.int32, (1, T), 1)
    sub = jax.lax.broadcasted_iota(jnp.int32, (T, 1), 0)

    for b in range(B):
        row0 = mi_ref[b * T:b * T + 1, :]
        c = (row0 != row0[:, 0:1]) & (lane >= 1)
        kidx = jnp.min(jnp.where(c, lane, 2 * T), axis=1, keepdims=True)
        featb = ft_ref[b * T:(b + 1) * T, :]
        sel = (sub == 0) | (sub == kidx)
        v_ref[b * T:(b + 1) * T, :] = jnp.where(sel, featb, 0.0)

    def step(j, alive):
        umj = um_ref[pl.ds(j, 1), :]
        new_alive = []
        for b in range(B):
            ab = alive[b] * umj[:, b:b + 1]
            row = mi_ref[pl.ds(b * T + j, 1), :]
            lmask = (lane < j) & (row == 1)
            lstar = jnp.max(jnp.where(lmask, lane, -1), axis=1, keepdims=True)
            has_l = lstar >= 0
            qwrow = qw_ref[pl.ds(b * T + j, 1), :]
            vb = v_ref[b * T:(b + 1) * T, :]
            s = jax.lax.dot_general(qwrow, vb, _CONTRACT_LAST,
                                    preferred_element_type=_F32)
            wm = (lane >= lstar) & (lane < j)
            sm = jnp.where(wm, s, -jnp.inf)
            m = jnp.max(sm, axis=1, keepdims=True)
            e = jnp.exp(sm - m)
            alpha = e / jnp.sum(e, axis=1, keepdims=True)
            vat = jnp.tanh(
                jax.lax.dot_general(alpha, vb, _CONTRACT_NATIVE,
                                    preferred_element_type=_F32))
            fr = ft_ref[pl.ds(b * T + j, 1), :]
            vcur = v_ref[pl.ds(b * T + j, 1), :]
            vj = jnp.where(has_l, vat, fr)
            vj = jnp.where(ab > 0, vj, vcur)
            v_ref[pl.ds(b * T + j, 1), :] = vj
            new_alive.append(ab)
        return tuple(new_alive)

    alive0 = tuple(jnp.ones((1, 1), _F32) for _ in range(B))
    jax.lax.fori_loop(1, T, step, alive0)

    o = jax.lax.dot_general(ft_ref[:], wo_ref[:], _CONTRACT_LAST,
                            preferred_element_type=_F32) + bo_ref[:]
    out_ref[:] = jnp.maximum(o * v_ref[:], 0.0) + ft_ref[:]


def kernel(feature, mask_intra, umask, W_init_trans, b_init_trans,
           W_qinter, b_qinter, W_attn, b_attn, W_out, b_out):
    del b_attn
    x2 = feature.reshape(B * T, D)
    mi2 = mask_intra.astype(jnp.int32).reshape(B * T, T)
    umt = umask.astype(_F32).T.reshape(T, B)
    bit = b_init_trans.reshape(1, D)
    bq = b_qinter.reshape(1, D)
    bo = b_out.reshape(1, D)
    out2 = pl.pallas_call(
        _sse_body,
        out_shape=jax.ShapeDtypeStruct((B * T, D), _F32),
        scratch_shapes=[pltpu.VMEM((B * T, D), _F32) for _ in range(3)],
    )(x2, mi2, umt, W_init_trans, bit, W_qinter, bq, W_attn, W_out, bo)
    return out2.reshape(B, T, D)


# speedup vs baseline: 2.8698x; 1.2627x over previous
import jax
import jax.numpy as jnp
from jax.experimental import pallas as pl
from jax.experimental.pallas import tpu as pltpu

B, T, D = 4, 128, 512
BS = 16
NBLK = T // BS
_F32 = jnp.float32
_NEG_INF = float("-inf")
_CONTRACT_LAST = (((1,), (1,)), ((), ()))
_CONTRACT_NATIVE = (((1,), (0,)), ((), ()))


def _dot(a, b, dims):
    return jax.lax.dot_general(a, b, dims, preferred_element_type=_F32)


def _sse_body(x_ref, mi_ref, um_ref, wit_ref, bit_ref, wq_ref, bq_ref,
              wa_ref, wo_ref, bo_ref, out_ref,
              v_ref, ft_ref, qw_ref, upre_ref, lstar_ref, mpre_ref, dpre_ref):
    x = x_ref[:]
    feat = jnp.maximum(_dot(x, wit_ref[:], _CONTRACT_LAST) + bit_ref[:], 0.0)
    ft_ref[:] = feat
    q = _dot(feat, wq_ref[:], _CONTRACT_LAST) + bq_ref[:]
    qw_ref[:] = q * wa_ref[:]

    lane = jax.lax.broadcasted_iota(jnp.int32, (1, T), 1)
    sub = jax.lax.broadcasted_iota(jnp.int32, (T, 1), 0)

    for b in range(B):
        row0 = mi_ref[b * T:b * T + 1, :]
        c = (row0 != row0[:, 0:1]) & (lane >= 1)
        kidx = jnp.min(jnp.where(c, lane, 2 * T), axis=1, keepdims=True)
        featb = ft_ref[b * T:(b + 1) * T, :]
        sel = (sub == 0) | (sub == kidx)
        v_ref[b * T:(b + 1) * T, :] = jnp.where(sel, featb, 0.0)

    alive = tuple(jnp.ones((1, 1), _F32) for _ in range(B))

    for k in range(NBLK):
        j0 = k * BS
        sub_bs = jax.lax.broadcasted_iota(jnp.int32, (BS, 1), 0) + j0

        for b in range(B):
            mi_blk = mi_ref[b * T + j0:b * T + j0 + BS, :]
            lmask = (lane < sub_bs) & (mi_blk == 1)
            lstar = jnp.max(jnp.where(lmask, lane, -1), axis=1,
                            keepdims=True)
            lstar_ref[b * BS:(b + 1) * BS, :] = lstar.astype(_F32)
            if j0 == 0:
                mpre_ref[b * BS:(b + 1) * BS, :] = jnp.full((BS, 1), _NEG_INF)
                dpre_ref[b * BS:(b + 1) * BS, :] = jnp.zeros((BS, 1), _F32)
                upre_ref[b * BS:(b + 1) * BS, :] = jnp.zeros((BS, D), _F32)
            else:
                vpre = v_ref[b * T:b * T + j0, :]
                qw_blk = qw_ref[b * T + j0:b * T + j0 + BS, :]
                s_pre = _dot(qw_blk, vpre, _CONTRACT_LAST)
                lane_pre = jax.lax.broadcasted_iota(jnp.int32, (1, j0), 1)
                wpre = lane_pre >= lstar
                m_pre = jnp.max(jnp.where(wpre, s_pre, _NEG_INF), axis=1,
                                keepdims=True)
                e_pre = jnp.where(wpre, jnp.exp(s_pre - m_pre), 0.0)
                mpre_ref[b * BS:(b + 1) * BS, :] = m_pre
                dpre_ref[b * BS:(b + 1) * BS, :] = jnp.sum(e_pre, axis=1,
                                                           keepdims=True)
                upre_ref[b * BS:(b + 1) * BS, :] = _dot(e_pre, vpre,
                                                        _CONTRACT_NATIVE)

        tlane = (jax.lax.broadcasted_iota(jnp.int32, (1, BS), 1) + j0)
        tlane_f = tlane.astype(_F32)

        def step(j, alive):
            jp = j - j0
            umj = um_ref[pl.ds(j, 1), :]
            new_alive = []
            for b in range(B):
                ab = alive[b] * umj[:, b:b + 1]
                sidx = b * BS + jp
                vidx = b * T + j
                lstar_row = lstar_ref[pl.ds(sidx, 1), :]
                m_pre = mpre_ref[pl.ds(sidx, 1), :]
                d_pre = dpre_ref[pl.ds(sidx, 1), :]
                u_pre = upre_ref[pl.ds(sidx, 1), :]
                v_blk = v_ref[b * T + j0:b * T + j0 + BS, :]
                qw_row = qw_ref[pl.ds(vidx, 1), :]
                s_in = _dot(qw_row, v_blk, _CONTRACT_LAST)
                wi = (tlane_f >= lstar_row) & (tlane < j)
                m_in = jnp.max(jnp.where(wi, s_in, _NEG_INF), axis=1,
                               keepdims=True)
                e_in = jnp.where(wi, jnp.exp(s_in - m_in), 0.0)
                d_in = jnp.sum(e_in, axis=1, keepdims=True)
                u_in = _dot(e_in, v_blk, _CONTRACT_NATIVE)
                m_tot = jnp.maximum(m_pre, m_in)
                c_pre = jnp.exp(m_pre - m_tot)
                c_in = jnp.exp(m_in - m_tot)
                num = c_pre * u_pre + c_in * u_in
                den = c_pre * d_pre + c_in * d_in
                v_att = jnp.tanh(num / den)
                fr = ft_ref[pl.ds(vidx, 1), :]
                vcur = v_ref[pl.ds(vidx, 1), :]
                vj = jnp.where(lstar_row >= 0.0, v_att, fr)
                vj = jnp.where(ab > 0, vj, vcur)
                v_ref[pl.ds(vidx, 1), :] = vj
                new_alive.append(ab)
            return tuple(new_alive)

        alive = jax.lax.fori_loop(max(j0, 1), j0 + BS, step, alive)

    o = _dot(ft_ref[:], wo_ref[:], _CONTRACT_LAST) + bo_ref[:]
    out_ref[:] = jnp.maximum(o * v_ref[:], 0.0) + ft_ref[:]


def kernel(feature, mask_intra, umask, W_init_trans, b_init_trans,
           W_qinter, b_qinter, W_attn, b_attn, W_out, b_out):
    del b_attn
    x2 = feature.reshape(B * T, D)
    mi2 = mask_intra.astype(jnp.int32).reshape(B * T, T)
    umt = umask.astype(_F32).T.reshape(T, B)
    bit = b_init_trans.reshape(1, D)
    bq = b_qinter.reshape(1, D)
    bo = b_out.reshape(1, D)
    out2 = pl.pallas_call(
        _sse_body,
        out_shape=jax.ShapeDtypeStruct((B * T, D), _F32),
        scratch_shapes=[
            pltpu.VMEM((B * T, D), _F32),
            pltpu.VMEM((B * T, D), _F32),
            pltpu.VMEM((B * T, D), _F32),
            pltpu.VMEM((B * BS, D), _F32),
            pltpu.VMEM((B * BS, 1), _F32),
            pltpu.VMEM((B * BS, 1), _F32),
            pltpu.VMEM((B * BS, 1), _F32),
        ],
    )(x2, mi2, umt, W_init_trans, bit, W_qinter, bq, W_attn, W_out, bo)
    return out2.reshape(B, T, D)


# speedup vs baseline: 8.4825x; 2.9558x over previous
import jax
import jax.numpy as jnp
from jax.experimental import pallas as pl
from jax.experimental.pallas import tpu as pltpu

B, T, D = 4, 128, 512
BS = 16
NBLK = T // BS
_F32 = jnp.float32
_NEG_INF = float("-inf")
_CONTRACT_LAST = (((1,), (1,)), ((), ()))
_CONTRACT_NATIVE = (((1,), (0,)), ((), ()))


def _dot(a, b, dims):
    return jax.lax.dot_general(a, b, dims, preferred_element_type=_F32)


def _sse_body(x_ref, mi_ref, um_ref, wit_ref, bit_ref, wq_ref, bq_ref,
              wa_ref, wo_ref, bo_ref, out_ref,
              v_ref, ft_ref, qw_ref, upre_ref, stat_ref):
    x = x_ref[:]
    feat = jnp.maximum(_dot(x, wit_ref[:], _CONTRACT_LAST) + bit_ref[:], 0.0)
    ft_ref[:] = feat
    q = _dot(feat, wq_ref[:], _CONTRACT_LAST) + bq_ref[:]
    qw_ref[:] = q * wa_ref[:]

    lane = jax.lax.broadcasted_iota(jnp.int32, (1, T), 1)
    sub = jax.lax.broadcasted_iota(jnp.int32, (T * B, 1), 0)
    tid = sub // B
    bcol = sub % B

    mi4 = mi_ref[0:B, :]
    c = (mi4 != mi4[:, 0:1]) & (lane >= 1)
    kidx4 = jnp.min(jnp.where(c, lane, 2 * T), axis=1, keepdims=True)
    sel = tid < 0
    for b in range(B):
        sel = sel | ((bcol == b) & ((tid == 0) | (tid == kidx4[b:b + 1])))
    v_ref[:] = jnp.where(sel, feat, 0.0)

    alive = jnp.ones((B, 1), _F32)
    sub_blk = jax.lax.broadcasted_iota(jnp.int32, (BS * B, 1), 0)
    lane_blk = jax.lax.broadcasted_iota(jnp.int32, (1, BS * B), 1)
    brow4 = jax.lax.broadcasted_iota(jnp.int32, (B, 1), 0)

    for k in range(NBLK):
        j0 = k * BS
        r0 = j0 * B

        mi_blk = mi_ref[r0:r0 + BS * B, :]
        jvec = sub_blk // B + j0
        lmask = (lane < jvec) & (mi_blk == 1)
        lstar = jnp.max(jnp.where(lmask, lane, -1), axis=1,
                        keepdims=True)
        if j0 == 0:
            stat_ref[:] = jnp.concatenate(
                [lstar.astype(_F32), jnp.full((BS * B, 1), _NEG_INF),
                 jnp.zeros((BS * B, 1), _F32)], axis=1)
            upre_ref[:] = jnp.zeros((BS * B, D), _F32)
        else:
            vpre = v_ref[0:r0, :]
            qw_blk = qw_ref[r0:r0 + BS * B, :]
            s_pre = _dot(qw_blk, vpre, _CONTRACT_LAST)
            lane_pre = jax.lax.broadcasted_iota(jnp.int32, (1, r0), 1)
            wpre = ((lane_pre % B == bcol[r0:r0 + BS * B, :])
                    & (lane_pre // B >= lstar))
            m_pre = jnp.max(jnp.where(wpre, s_pre, _NEG_INF), axis=1,
                            keepdims=True)
            e_pre = jnp.where(wpre, jnp.exp(s_pre - m_pre), 0.0)
            stat_ref[:] = jnp.concatenate(
                [lstar.astype(_F32), m_pre,
                 jnp.sum(e_pre, axis=1, keepdims=True)], axis=1)
            upre_ref[:] = _dot(e_pre, vpre, _CONTRACT_NATIVE)

        tc = lane_blk // B + j0
        bc = lane_blk % B

        def cat4(ref, base):
            return jnp.concatenate(
                [ref[pl.ds(base + i, 1), :] for i in range(B)], axis=0)

        def step(j, alive):
            rp = (j - j0) * B
            v_blk = v_ref[r0:r0 + BS * B, :]
            qw4 = cat4(qw_ref, B * j)
            s_in = _dot(qw4, v_blk, _CONTRACT_LAST)
            stat4 = cat4(stat_ref, rp)
            lstar4 = stat4[:, 0:1]
            m_pre = stat4[:, 1:2]
            d_pre = stat4[:, 2:3]
            wi = (bc == brow4) & (tc.astype(_F32) >= lstar4) & (tc < j)
            m_in = jnp.max(jnp.where(wi, s_in, _NEG_INF), axis=1,
                           keepdims=True)
            e_in = jnp.where(wi, jnp.exp(s_in - m_in), 0.0)
            d_in = jnp.sum(e_in, axis=1, keepdims=True)
            u_in = _dot(e_in, v_blk, _CONTRACT_NATIVE)
            u_pre = cat4(upre_ref, rp)
            m_tot = jnp.maximum(m_pre, m_in)
            c_pre = jnp.exp(m_pre - m_tot)
            c_in = jnp.exp(m_in - m_tot)
            num = c_pre * u_pre + c_in * u_in
            den = c_pre * d_pre + c_in * d_in
            v_att = jnp.tanh(num / den)
            fr = cat4(ft_ref, B * j)
            alive = alive * cat4(um_ref, B * j)
            vj = jnp.where(lstar4 >= 0.0, v_att, fr)
            vj = jnp.where(alive > 0, vj, jnp.where(kidx4 == j, fr, 0.0))
            for i in range(B):
                v_ref[pl.ds(B * j + i, 1), :] = vj[i:i + 1, :]
            return alive

        alive = jax.lax.fori_loop(max(j0, 1), j0 + BS, step, alive)

    o = _dot(ft_ref[:], wo_ref[:], _CONTRACT_LAST) + bo_ref[:]
    out_ref[:] = jnp.maximum(o * v_ref[:], 0.0) + ft_ref[:]


def kernel(feature, mask_intra, umask, W_init_trans, b_init_trans,
           W_qinter, b_qinter, W_attn, b_attn, W_out, b_out):
    del b_attn
    x2 = feature.transpose(1, 0, 2).reshape(T * B, D)
    mi2 = mask_intra.astype(jnp.int32).transpose(1, 0, 2).reshape(T * B, T)
    umr = umask.astype(_F32).T.reshape(T * B, 1)
    bit = b_init_trans.reshape(1, D)
    bq = b_qinter.reshape(1, D)
    bo = b_out.reshape(1, D)
    out2 = pl.pallas_call(
        _sse_body,
        out_shape=jax.ShapeDtypeStruct((T * B, D), _F32),
        scratch_shapes=[
            pltpu.VMEM((T * B, D), _F32),
            pltpu.VMEM((T * B, D), _F32),
            pltpu.VMEM((T * B, D), _F32),
            pltpu.VMEM((BS * B, D), _F32),
            pltpu.VMEM((BS * B, 3), _F32),
        ],
    )(x2, mi2, umr, W_init_trans, bit, W_qinter, bq, W_attn, W_out, bo)
    return out2.reshape(T, B, D).transpose(1, 0, 2)


# speedup vs baseline: 9.1329x; 1.0767x over previous
import jax
import jax.numpy as jnp
from jax.experimental import pallas as pl
from jax.experimental.pallas import tpu as pltpu

B, T, D = 4, 128, 512
BS = 16
NBLK = T // BS
_F32 = jnp.float32
_NEG_INF = float("-inf")
_CONTRACT_LAST = (((1,), (1,)), ((), ()))
_CONTRACT_NATIVE = (((1,), (0,)), ((), ()))


def _dot(a, b, dims):
    return jax.lax.dot_general(a, b, dims, preferred_element_type=_F32)


def _sse_body(x_ref, mi_ref, um_ref, wit_ref, bit_ref, wq_ref, bq_ref,
              wa_ref, wo_ref, bo_ref, out_ref,
              v_ref, ft_ref, qw_ref, upre_ref, stat_ref):
    x = x_ref[:]
    feat = jnp.maximum(_dot(x, wit_ref[:], _CONTRACT_LAST) + bit_ref[:], 0.0)
    ft_ref[:] = feat
    q = _dot(feat, wq_ref[:], _CONTRACT_LAST) + bq_ref[:]
    qw_ref[:] = q * wa_ref[:]

    lane = jax.lax.broadcasted_iota(jnp.int32, (1, T), 1)
    sub = jax.lax.broadcasted_iota(jnp.int32, (T * B, 1), 0)
    tid = sub // B
    bcol = sub % B

    mi4 = mi_ref[0:B, :]
    c = (mi4 != mi4[:, 0:1]) & (lane >= 1)
    kidx4 = jnp.min(jnp.where(c, lane, 2 * T), axis=1, keepdims=True)
    sel = tid < 0
    for b in range(B):
        sel = sel | ((bcol == b) & ((tid == 0) | (tid == kidx4[b:b + 1])))
    v_ref[:] = jnp.where(sel, feat, 0.0)

    alive = jnp.ones((B, 1), _F32)
    sub_blk = jax.lax.broadcasted_iota(jnp.int32, (BS * B, 1), 0)
    lane_blk = jax.lax.broadcasted_iota(jnp.int32, (1, BS * B), 1)
    brow4 = jax.lax.broadcasted_iota(jnp.int32, (B, 1), 0)

    def cat4(ref, base):
        return jnp.concatenate(
            [ref[pl.ds(base + i, 1), :] for i in range(B)], axis=0)

    vprev = cat4(v_ref, 0)

    for k in range(NBLK):
        j0 = k * BS
        r0 = j0 * B
        lo = max(j0, 1)

        mi_blk = mi_ref[r0:r0 + BS * B, :]
        jvec = sub_blk // B + j0
        lmask = (lane < jvec) & (mi_blk == 1)
        lstar = jnp.max(jnp.where(lmask, lane, -1), axis=1,
                        keepdims=True)
        if j0 == 0:
            stat_ref[:] = jnp.concatenate(
                [lstar.astype(_F32), jnp.full((BS * B, 1), _NEG_INF),
                 jnp.zeros((BS * B, 1), _F32)], axis=1)
            upre_ref[:] = jnp.zeros((BS * B, D), _F32)
        else:
            vpre = v_ref[0:r0, :]
            qw_blk = qw_ref[r0:r0 + BS * B, :]
            s_pre = _dot(qw_blk, vpre, _CONTRACT_LAST)
            lane_pre = jax.lax.broadcasted_iota(jnp.int32, (1, r0), 1)
            wpre = ((lane_pre % B == bcol[r0:r0 + BS * B, :])
                    & (lane_pre // B >= lstar))
            m_pre = jnp.max(jnp.where(wpre, s_pre, _NEG_INF), axis=1,
                            keepdims=True)
            e_pre = jnp.where(wpre, jnp.exp(s_pre - m_pre), 0.0)
            stat_ref[:] = jnp.concatenate(
                [lstar.astype(_F32), m_pre,
                 jnp.sum(e_pre, axis=1, keepdims=True)], axis=1)
            upre_ref[:] = _dot(e_pre, vpre, _CONTRACT_NATIVE)

        tc = lane_blk // B + j0
        tc_f = tc.astype(_F32)
        bc = lane_blk % B

        def step(j, carry):
            alive, vprev, m_ip, d_ip, u_ip, stats_cur = carry
            rp = (j - j0) * B
            lstar4 = stats_cur[:, 0:1]
            m_pre = stats_cur[:, 1:2]
            d_pre = stats_cur[:, 2:3]
            qw_cur = cat4(qw_ref, B * j)
            v_blk = v_ref[r0:r0 + BS * B, :]
            u_pre = cat4(upre_ref, rp)

            corr_s = jnp.sum(qw_cur * vprev, axis=1, keepdims=True)
            corr_m = jnp.where(j > j0, corr_s, _NEG_INF)
            m_all = jnp.maximum(jnp.maximum(m_ip, corr_m), m_pre)
            c_pre = jnp.exp(m_pre - m_all)
            c_ip = jnp.exp(m_ip - m_all)
            e_c = jnp.exp(corr_m - m_all)
            num = c_pre * u_pre + c_ip * u_ip + e_c * vprev
            den = c_pre * d_pre + c_ip * d_ip + e_c
            v_att = jnp.tanh(num / den)
            fr = cat4(ft_ref, B * j)
            alive = alive * cat4(um_ref, B * j)
            vj = jnp.where(lstar4 >= 0.0, v_att, fr)
            vj = jnp.where(alive > 0, vj, jnp.where(kidx4 == j, fr, 0.0))
            for i in range(B):
                v_ref[pl.ds(B * j + i, 1), :] = vj[i:i + 1, :]

            jn = jnp.minimum(j + 1, T - 1)
            qw_next = cat4(qw_ref, B * jn)
            s_next = _dot(qw_next, v_blk, _CONTRACT_LAST)
            rpn = jnp.minimum(rp + B, (BS - 1) * B)
            stats_next = cat4(stat_ref, rpn)
            lstar_n = stats_next[:, 0:1]
            wip = (bc == brow4) & (tc_f >= lstar_n) & (tc < j)
            m_ip_n = jnp.max(jnp.where(wip, s_next, _NEG_INF), axis=1,
                             keepdims=True)
            e_ip = jnp.where(wip, jnp.exp(s_next - m_ip_n), 0.0)
            d_ip_n = jnp.sum(e_ip, axis=1, keepdims=True)
            u_ip_n = _dot(e_ip, v_blk, _CONTRACT_NATIVE)
            return (alive, vj, m_ip_n, d_ip_n, u_ip_n, stats_next)

        carry = (alive, vprev,
                 jnp.full((B, 1), _NEG_INF, _F32),
                 jnp.zeros((B, 1), _F32),
                 jnp.zeros((B, D), _F32),
                 cat4(stat_ref, (lo - j0) * B))
        carry = jax.lax.fori_loop(lo, j0 + BS, step, carry)
        alive, vprev = carry[0], carry[1]

    o = _dot(ft_ref[:], wo_ref[:], _CONTRACT_LAST) + bo_ref[:]
    out_ref[:] = jnp.maximum(o * v_ref[:], 0.0) + ft_ref[:]


def kernel(feature, mask_intra, umask, W_init_trans, b_init_trans,
           W_qinter, b_qinter, W_attn, b_attn, W_out, b_out):
    del b_attn
    x2 = feature.transpose(1, 0, 2).reshape(T * B, D)
    mi2 = mask_intra.astype(jnp.int32).transpose(1, 0, 2).reshape(T * B, T)
    umr = umask.astype(_F32).T.reshape(T * B, 1)
    bit = b_init_trans.reshape(1, D)
    bq = b_qinter.reshape(1, D)
    bo = b_out.reshape(1, D)
    out2 = pl.pallas_call(
        _sse_body,
        out_shape=jax.ShapeDtypeStruct((T * B, D), _F32),
        scratch_shapes=[
            pltpu.VMEM((T * B, D), _F32),
            pltpu.VMEM((T * B, D), _F32),
            pltpu.VMEM((T * B, D), _F32),
            pltpu.VMEM((BS * B, D), _F32),
            pltpu.VMEM((BS * B, 3), _F32),
        ],
    )(x2, mi2, umr, W_init_trans, bit, W_qinter, bq, W_attn, W_out, bo)
    return out2.reshape(T, B, D).transpose(1, 0, 2)


# speedup vs baseline: 10.4378x; 1.1429x over previous
import jax
import jax.numpy as jnp
from jax.experimental import pallas as pl
from jax.experimental.pallas import tpu as pltpu

B, T, D = 4, 128, 512
BS = 16
NBLK = T // BS
_F32 = jnp.float32
_NEG_INF = float("-inf")
_CONTRACT_LAST = (((1,), (1,)), ((), ()))
_CONTRACT_NATIVE = (((1,), (0,)), ((), ()))


def _dot(a, b, dims):
    return jax.lax.dot_general(a, b, dims, preferred_element_type=_F32)


def _sse_body(x_ref, mi_ref, um_ref, wit_ref, bit_ref, wq_ref, bq_ref,
              wa_ref, wo_ref, bo_ref, out_ref,
              v_ref, ft_ref, qw_ref, upre_ref, stat_ref):
    x = x_ref[:]
    feat = jnp.maximum(_dot(x, wit_ref[:], _CONTRACT_LAST) + bit_ref[:], 0.0)
    ft_ref[:] = feat
    q = _dot(feat, wq_ref[:], _CONTRACT_LAST) + bq_ref[:]
    qw_ref[:] = q * wa_ref[:]

    lane = jax.lax.broadcasted_iota(jnp.int32, (1, T), 1)
    sub = jax.lax.broadcasted_iota(jnp.int32, (T * B, 1), 0)
    tid = sub // B
    bcol = sub % B

    mi4 = mi_ref[0:B, :]
    c = (mi4 != mi4[:, 0:1]) & (lane >= 1)
    kidx4 = jnp.min(jnp.where(c, lane, 2 * T), axis=1, keepdims=True)
    sel = tid < 0
    for b in range(B):
        sel = sel | ((bcol == b) & ((tid == 0) | (tid == kidx4[b:b + 1])))
    v_ref[:] = jnp.where(sel, feat, 0.0)

    alive = jnp.ones((B, 1), _F32)
    sub_blk = jax.lax.broadcasted_iota(jnp.int32, (BS * B, 1), 0)
    lane_blk = jax.lax.broadcasted_iota(jnp.int32, (1, BS * B), 1)
    brow4 = jax.lax.broadcasted_iota(jnp.int32, (B, 1), 0)

    def cat4(ref, base):
        return jnp.concatenate(
            [ref[pl.ds(base + i, 1), :] for i in range(B)], axis=0)

    vprev = cat4(v_ref, 0)

    for k in range(NBLK):
        j0 = k * BS
        r0 = j0 * B
        lo = max(j0, 1)

        mi_blk = mi_ref[r0:r0 + BS * B, :]
        jvec = sub_blk // B + j0
        lmask = (lane < jvec) & (mi_blk == 1)
        lstar = jnp.max(jnp.where(lmask, lane, -1), axis=1,
                        keepdims=True)
        if j0 == 0:
            stat_ref[:] = jnp.concatenate(
                [lstar.astype(_F32), jnp.full((BS * B, 1), _NEG_INF),
                 jnp.zeros((BS * B, 1), _F32)], axis=1)
            upre_ref[:] = jnp.zeros((BS * B, D), _F32)
        else:
            vpre = v_ref[0:r0, :]
            qw_blk = qw_ref[r0:r0 + BS * B, :]
            s_pre = _dot(qw_blk, vpre, _CONTRACT_LAST)
            lane_pre = jax.lax.broadcasted_iota(jnp.int32, (1, r0), 1)
            wpre = ((lane_pre % B == bcol[r0:r0 + BS * B, :])
                    & (lane_pre // B >= lstar))
            m_pre = jnp.max(jnp.where(wpre, s_pre, _NEG_INF), axis=1,
                            keepdims=True)
            e_pre = jnp.where(wpre, jnp.exp(s_pre - m_pre), 0.0)
            stat_ref[:] = jnp.concatenate(
                [lstar.astype(_F32), m_pre,
                 jnp.sum(e_pre, axis=1, keepdims=True)], axis=1)
            upre_ref[:] = _dot(e_pre, vpre, _CONTRACT_NATIVE)

        tc = lane_blk // B + j0
        tc_f = tc.astype(_F32)
        bc = lane_blk % B

        def step(j, carry):
            alive, vprev, m_ip, d_ip, e_ip, stats_cur, qw_cur = carry
            rp = (j - j0) * B
            v_blk = v_ref[r0:r0 + BS * B, :]
            u_ip = _dot(e_ip, v_blk, _CONTRACT_NATIVE)
            jn = jnp.minimum(j + 1, T - 1)
            qw_next = cat4(qw_ref, B * jn)
            s_next = _dot(qw_next, v_blk, _CONTRACT_LAST)

            u_pre = cat4(upre_ref, rp)
            fr = cat4(ft_ref, B * j)
            umv = cat4(um_ref, B * j)
            rpn = jnp.minimum(rp + B, (BS - 1) * B)
            stats_next = cat4(stat_ref, rpn)
            lstar4 = stats_cur[:, 0:1]
            m_pre = stats_cur[:, 1:2]
            d_pre = stats_cur[:, 2:3]

            corr_s = jnp.sum(qw_cur * vprev, axis=1, keepdims=True)
            corr_m = jnp.where(j > j0, corr_s, _NEG_INF)
            m_all = jnp.maximum(jnp.maximum(m_ip, corr_m), m_pre)
            c_pre = jnp.exp(m_pre - m_all)
            c_ip = jnp.exp(m_ip - m_all)
            e_c = jnp.exp(corr_m - m_all)
            num = c_pre * u_pre + c_ip * u_ip + e_c * vprev
            den = c_pre * d_pre + c_ip * d_ip + e_c
            v_att = jnp.tanh(num / den)
            alive = alive * umv
            vj = jnp.where(lstar4 >= 0.0, v_att, fr)
            vj = jnp.where(alive > 0, vj, jnp.where(kidx4 == j, fr, 0.0))
            for i in range(B):
                v_ref[pl.ds(B * j + i, 1), :] = vj[i:i + 1, :]

            lstar_n = stats_next[:, 0:1]
            wip = (bc == brow4) & (tc_f >= lstar_n) & (tc < j)
            m_ip_n = jnp.max(jnp.where(wip, s_next, _NEG_INF), axis=1,
                             keepdims=True)
            e_ip_n = jnp.where(wip, jnp.exp(s_next - m_ip_n), 0.0)
            d_ip_n = jnp.sum(e_ip_n, axis=1, keepdims=True)
            return (alive, vj, m_ip_n, d_ip_n, e_ip_n, stats_next, qw_next)

        carry = (alive, vprev,
                 jnp.full((B, 1), _NEG_INF, _F32),
                 jnp.zeros((B, 1), _F32),
                 jnp.zeros((B, BS * B), _F32),
                 cat4(stat_ref, (lo - j0) * B),
                 cat4(qw_ref, B * lo))
        carry = jax.lax.fori_loop(lo, j0 + BS, step, carry)
        alive, vprev = carry[0], carry[1]

    o = _dot(ft_ref[:], wo_ref[:], _CONTRACT_LAST) + bo_ref[:]
    out_ref[:] = jnp.maximum(o * v_ref[:], 0.0) + ft_ref[:]


def kernel(feature, mask_intra, umask, W_init_trans, b_init_trans,
           W_qinter, b_qinter, W_attn, b_attn, W_out, b_out):
    del b_attn
    x2 = feature.transpose(1, 0, 2).reshape(T * B, D)
    mi2 = mask_intra.astype(jnp.int32).transpose(1, 0, 2).reshape(T * B, T)
    umr = umask.astype(_F32).T.reshape(T * B, 1)
    bit = b_init_trans.reshape(1, D)
    bq = b_qinter.reshape(1, D)
    bo = b_out.reshape(1, D)
    out2 = pl.pallas_call(
        _sse_body,
        out_shape=jax.ShapeDtypeStruct((T * B, D), _F32),
        scratch_shapes=[
            pltpu.VMEM((T * B, D), _F32),
            pltpu.VMEM((T * B, D), _F32),
            pltpu.VMEM((T * B, D), _F32),
            pltpu.VMEM((BS * B, D), _F32),
            pltpu.VMEM((BS * B, 3), _F32),
        ],
    )(x2, mi2, umr, W_init_trans, bit, W_qinter, bq, W_attn, W_out, bo)
    return out2.reshape(T, B, D).transpose(1, 0, 2)
